# speedup vs baseline: 2.8344x; 2.8344x over previous
import jax
import jax.numpy as jnp
from jax.experimental import pallas as pl
from jax.experimental.pallas import tpu as pltpu

N_KV_HEADS = 8
MAX_CONTEXT = 8192
HEAD_DIM = 128
Q_LEN = 32

CHUNK = 512


def _update_body(pos_ref, kc_ref, vc_ref, k_ref, v_ref, ko_ref, vo_ref):
    ko_ref[...] = kc_ref[...]
    vo_ref[...] = vc_ref[...]
    base = pl.program_id(0) * CHUNK
    for i in range(Q_LEN):
        p = pos_ref[i]
        rel = p - base

        @pl.when((rel >= 0) & (rel < CHUNK))
        def _():
            ko_ref[:, :, pl.ds(rel, 1), :] = k_ref[:, :, pl.ds(i, 1), :]
            vo_ref[:, :, pl.ds(rel, 1), :] = v_ref[:, :, pl.ds(i, 1), :]


def kernel(k_cache, v_cache, pos_ids, k, v):
    pos = pos_ids.astype(jnp.int32)
    cache_spec = pl.BlockSpec(
        (1, N_KV_HEADS, CHUNK, HEAD_DIM), lambda i, pos_ref: (0, 0, i, 0)
    )
    new_spec = pl.BlockSpec(
        (1, N_KV_HEADS, Q_LEN, HEAD_DIM), lambda i, pos_ref: (0, 0, 0, 0)
    )
    out_shape = jax.ShapeDtypeStruct(k_cache.shape, k_cache.dtype)
    grid_spec = pltpu.PrefetchScalarGridSpec(
        num_scalar_prefetch=1,
        grid=(MAX_CONTEXT // CHUNK,),
        in_specs=[cache_spec, cache_spec, new_spec, new_spec],
        out_specs=[cache_spec, cache_spec],
    )
    kout, vout = pl.pallas_call(
        _update_body,
        grid_spec=grid_spec,
        out_shape=[out_shape, out_shape],
    )(pos, k_cache, v_cache, k, v)
    return (kout, vout)
